# baseline (speedup 1.0000x reference)
import functools

import jax
import jax.numpy as jnp
from jax import lax
from jax.experimental import pallas as pl
from jax.experimental.pallas import tpu as pltpu
from jax.experimental.pallas import tpu_sc as plsc

B = 1024
L = 50
RDIM = 64
TDIM = 64
D = RDIM + TDIM
H = 256
OUT = 128

NCHUNK = 5
LC = L // NCHUNK
CB = LC * B

_NC = 2
_NS = 16
_NW = _NC * _NS
_BW = B // _NW
_TPW = _BW * LC
_CH = 80
_NCH = _TPW // _CH


def _sc_gather_chunk(chunk, seq, tsq, rtab, ttab):
    mesh = plsc.VectorSubcoreMesh(core_axis_name="c", subcore_axis_name="s")
    l0 = chunk * LC

    @functools.partial(
        pl.kernel,
        out_type=jax.ShapeDtypeStruct((CB, D), jnp.float32),
        mesh=mesh,
        scratch_types=(
            pltpu.VMEM((_BW, 128), jnp.int32),
            pltpu.VMEM((_BW, 128), jnp.float32),
            pltpu.VMEM((_NCH, _CH), jnp.int32),
            pltpu.VMEM((_TPW,), jnp.float32),
            pltpu.VMEM((_NCH, _CH), jnp.int32),
            pltpu.VMEM((_TPW, RDIM), jnp.float32),
            pltpu.VMEM((_TPW, TDIM), jnp.float32),
            pltpu.VMEM((_TPW, D), jnp.float32),
            pltpu.SemaphoreType.DMA,
        ),
        compiler_params=pltpu.CompilerParams(use_tc_tiling_on_sc=False,
                                            needs_layout_passes=False),
    )
    def k(seq_hbm, tsq_hbm, rtab_hbm, ttab_hbm, out_hbm,
          blk_i, blk_t, idxp, tp, destp, rbuf, tbuf, wide, sem):
        wid = lax.axis_index("s") * _NC + lax.axis_index("c")
        b0 = wid * _BW
        pltpu.sync_copy(seq_hbm.at[pl.ds(b0, _BW)], blk_i)
        pltpu.sync_copy(tsq_hbm.at[pl.ds(b0, _BW)], blk_t)

        for g in range(_TPW // 16):
            j = g * 16 + lax.iota(jnp.int32, 16)
            bl = jnp.bitwise_and(j, _BW - 1)
            kk = jnp.right_shift(j, 5)
            ids = plsc.load_gather(blk_i, [bl, kk + l0])
            tvs = plsc.load_gather(blk_t, [bl, kk + l0])
            row, col = (g * 16) // _CH, (g * 16) % _CH
            idxp[row, pl.ds(col, 16)] = ids
            destp[row, pl.ds(col, 16)] = kk * B + (b0 + bl)
            tp[pl.ds(g * 16, 16)] = tvs

        descs = []
        for q in range(_NCH):
            descs.append(pltpu.async_copy(
                rtab_hbm.at[idxp.at[q]],
                rbuf.at[pl.ds(q * _CH, _CH)], sem))
            descs.append(pltpu.async_copy(
                ttab_hbm.at[idxp.at[q]],
                tbuf.at[pl.ds(q * _CH, _CH)], sem))
        for dsc in descs:
            dsc.wait()

        def row_body(i, _):
            tvec = plsc.load_gather(tp, [i + jnp.zeros((16,), jnp.int32)])
            for q4 in range(RDIM // 16):
                sl = pl.ds(q4 * 16, 16)
                wide[i, sl] = rbuf[i, sl]
                wide[i, pl.ds(RDIM + q4 * 16, 16)] = tbuf[i, sl] * tvec
            return 0

        lax.fori_loop(0, _TPW, row_body, 0)

        descs2 = [
            pltpu.async_copy(
                wide.at[pl.ds(q * _CH, _CH)],
                out_hbm.at[destp.at[q]], sem)
            for q in range(_NCH)
        ]
        for dsc in descs2:
            dsc.wait()

    return k(seq, tsq, rtab, ttab)


def _make_lstm_body(last):
    def body(x_ref, wih_ref, whh_ref, b_ref, fcw_ref, fcb_ref,
             h0_ref, c0_ref, ho_ref, co_ref, out_ref, h_scr, c_scr):
        t = pl.program_id(0)

        @pl.when(t == 0)
        def _():
            h_scr[...] = h0_ref[...]
            c_scr[...] = c0_ref[...]

        xs = x_ref[0].astype(jnp.bfloat16)
        h = h_scr[...]
        gates = (
            jnp.dot(xs, wih_ref[...], preferred_element_type=jnp.float32)
            + jnp.dot(h.astype(jnp.bfloat16), whh_ref[...],
                      preferred_element_type=jnp.float32)
            + b_ref[...]
        )
        th = jnp.tanh(gates)
        i = th[:, :H] * 0.5 + 0.5
        f = th[:, H:2 * H] * 0.5 + 0.5
        g = th[:, 2 * H:3 * H]
        o = th[:, 3 * H:] * 0.5 + 0.5
        c = f * c_scr[...] + i * g
        hn = o * jnp.tanh(c)
        c_scr[...] = c
        h_scr[...] = hn

        @pl.when(t == LC - 1)
        def _():
            ho_ref[...] = hn
            co_ref[...] = c
            if last:
                out_ref[...] = jnp.tanh(
                    jnp.dot(hn.astype(jnp.bfloat16), fcw_ref[...],
                            preferred_element_type=jnp.float32)
                    + fcb_ref[...]
                ) * 0.5 + 0.5

    return body


def _lstm_tc(x, wih_t, whh_t, bias, fcw_t, fcb, h0, c0, last):
    full = lambda t: (0, 0)
    return pl.pallas_call(
        _make_lstm_body(last),
        grid=(LC,),
        in_specs=[
            pl.BlockSpec((1, B, D), lambda t: (t, 0, 0)),
            pl.BlockSpec((D, 4 * H), full),
            pl.BlockSpec((H, 4 * H), full),
            pl.BlockSpec((1, 4 * H), full),
            pl.BlockSpec((H, OUT), full),
            pl.BlockSpec((1, OUT), full),
            pl.BlockSpec((B, H), full),
            pl.BlockSpec((B, H), full),
        ],
        out_specs=[
            pl.BlockSpec((B, H), full),
            pl.BlockSpec((B, H), full),
            pl.BlockSpec((B, OUT), full),
        ],
        out_shape=[
            jax.ShapeDtypeStruct((B, H), jnp.float32),
            jax.ShapeDtypeStruct((B, H), jnp.float32),
            jax.ShapeDtypeStruct((B, OUT), jnp.float32),
        ],
        scratch_shapes=[
            pltpu.VMEM((B, H), jnp.float32),
            pltpu.VMEM((B, H), jnp.float32),
        ],
    )(x, wih_t, whh_t, bias, fcw_t, fcb, h0, c0)


def kernel(region_sequences, time_sequences, region_table, time_table,
           W_ih, W_hh, b_ih, b_hh, fc_W, fc_b):
    colscale = jnp.concatenate(
        [jnp.full((2 * H,), 0.5, jnp.float32),
         jnp.ones((H,), jnp.float32),
         jnp.full((H,), 0.5, jnp.float32)]
    )
    wih_t = (W_ih.T * colscale[None, :]).astype(jnp.bfloat16)
    whh_t = (W_hh.T * colscale[None, :]).astype(jnp.bfloat16)
    bias = ((b_ih + b_hh) * colscale).reshape(1, 4 * H)
    fcw_t = (fc_W.T * 0.5).astype(jnp.bfloat16)
    fcb = (fc_b * 0.5).reshape(1, OUT)

    seq_p = jnp.pad(region_sequences, ((0, 0), (0, 128 - L)))
    tsq_p = jnp.pad(time_sequences, ((0, 0), (0, 128 - L)))
    xs = [_sc_gather_chunk(c, seq_p, tsq_p,
                           region_table, time_table).reshape(LC, B, D)
          for c in range(NCHUNK)]
    h = jnp.zeros((B, H), jnp.float32)
    c = jnp.zeros((B, H), jnp.float32)
    out = None
    for ci in range(NCHUNK):
        h, c, out = _lstm_tc(xs[ci], wih_t, whh_t, bias, fcw_t, fcb,
                             h, c, last=(ci == NCHUNK - 1))
    return out


# speedup vs baseline: 1.1468x; 1.1468x over previous
import functools

import jax
import jax.numpy as jnp
from jax import lax
from jax.experimental import pallas as pl
from jax.experimental.pallas import tpu as pltpu
from jax.experimental.pallas import tpu_sc as plsc

B = 1024
L = 50
RDIM = 64
TDIM = 64
D = RDIM + TDIM
H = 256
OUT = 128

NCHUNK = 5
LC = L // NCHUNK
CB = LC * B

_NC = 2
_NS = 16
_NW = _NC * _NS
_BW = B // _NW
_TPW = _BW * LC
_CH = 80
_NCH = _TPW // _CH


def _sc_gather_chunk(chunk, seq, tsq, ctab):
    mesh = plsc.VectorSubcoreMesh(core_axis_name="c", subcore_axis_name="s")
    l0 = chunk * LC

    @functools.partial(
        pl.kernel,
        out_type=jax.ShapeDtypeStruct((CB, D), jnp.float32),
        mesh=mesh,
        scratch_types=(
            pltpu.VMEM((_BW, 128), jnp.int32),
            pltpu.VMEM((_BW, 128), jnp.float32),
            pltpu.VMEM((_NCH, _CH), jnp.int32),
            pltpu.VMEM((_TPW,), jnp.float32),
            pltpu.VMEM((_NCH, _CH), jnp.int32),
            pltpu.VMEM((_TPW, D), jnp.float32),
            pltpu.SemaphoreType.DMA,
        ),
        compiler_params=pltpu.CompilerParams(use_tc_tiling_on_sc=False,
                                            needs_layout_passes=False),
    )
    def k(seq_hbm, tsq_hbm, ctab_hbm, out_hbm,
          blk_i, blk_t, idxp, tp, destp, wide, sem):
        wid = lax.axis_index("s") * _NC + lax.axis_index("c")
        b0 = wid * _BW
        pltpu.sync_copy(seq_hbm.at[pl.ds(b0, _BW)], blk_i)
        pltpu.sync_copy(tsq_hbm.at[pl.ds(b0, _BW)], blk_t)

        for g in range(_TPW // 16):
            j = g * 16 + lax.iota(jnp.int32, 16)
            bl = jnp.bitwise_and(j, _BW - 1)
            kk = jnp.right_shift(j, 5)
            ids = plsc.load_gather(blk_i, [bl, kk + l0])
            tvs = plsc.load_gather(blk_t, [bl, kk + l0])
            row, col = (g * 16) // _CH, (g * 16) % _CH
            idxp[row, pl.ds(col, 16)] = ids
            destp[row, pl.ds(col, 16)] = kk * B + (b0 + bl)
            tp[pl.ds(g * 16, 16)] = tvs

        descs = [
            pltpu.async_copy(
                ctab_hbm.at[idxp.at[q]],
                wide.at[pl.ds(q * _CH, _CH)], sem)
            for q in range(_NCH)
        ]
        for dsc in descs:
            dsc.wait()

        def row_body(i, _):
            tvec = plsc.load_gather(tp, [i + jnp.zeros((16,), jnp.int32)])
            for q4 in range(TDIM // 16):
                sl = pl.ds(RDIM + q4 * 16, 16)
                wide[i, sl] = wide[i, sl] * tvec
            return 0

        lax.fori_loop(0, _TPW, row_body, 0)

        descs2 = [
            pltpu.async_copy(
                wide.at[pl.ds(q * _CH, _CH)],
                out_hbm.at[destp.at[q]], sem)
            for q in range(_NCH)
        ]
        for dsc in descs2:
            dsc.wait()

    return k(seq, tsq, ctab)


def _make_lstm_body(last):
    def body(x_ref, wih_ref, whh_ref, b_ref, fcw_ref, fcb_ref,
             h0_ref, c0_ref, ho_ref, co_ref, out_ref, h_scr, c_scr):
        t = pl.program_id(0)

        @pl.when(t == 0)
        def _():
            h_scr[...] = h0_ref[...]
            c_scr[...] = c0_ref[...]

        xs = x_ref[0].astype(jnp.bfloat16)
        h = h_scr[...]
        gates = (
            jnp.dot(xs, wih_ref[...], preferred_element_type=jnp.float32)
            + jnp.dot(h.astype(jnp.bfloat16), whh_ref[...],
                      preferred_element_type=jnp.float32)
            + b_ref[...]
        )
        th = jnp.tanh(gates)
        i = th[:, :H] * 0.5 + 0.5
        f = th[:, H:2 * H] * 0.5 + 0.5
        g = th[:, 2 * H:3 * H]
        o = th[:, 3 * H:] * 0.5 + 0.5
        c = f * c_scr[...] + i * g
        hn = o * jnp.tanh(c)
        c_scr[...] = c
        h_scr[...] = hn

        @pl.when(t == LC - 1)
        def _():
            ho_ref[...] = hn
            co_ref[...] = c
            if last:
                out_ref[...] = jnp.tanh(
                    jnp.dot(hn.astype(jnp.bfloat16), fcw_ref[...],
                            preferred_element_type=jnp.float32)
                    + fcb_ref[...]
                ) * 0.5 + 0.5

    return body


def _lstm_tc(x, wih_t, whh_t, bias, fcw_t, fcb, h0, c0, last):
    full = lambda t: (0, 0)
    return pl.pallas_call(
        _make_lstm_body(last),
        grid=(LC,),
        in_specs=[
            pl.BlockSpec((1, B, D), lambda t: (t, 0, 0)),
            pl.BlockSpec((D, 4 * H), full),
            pl.BlockSpec((H, 4 * H), full),
            pl.BlockSpec((1, 4 * H), full),
            pl.BlockSpec((H, OUT), full),
            pl.BlockSpec((1, OUT), full),
            pl.BlockSpec((B, H), full),
            pl.BlockSpec((B, H), full),
        ],
        out_specs=[
            pl.BlockSpec((B, H), full),
            pl.BlockSpec((B, H), full),
            pl.BlockSpec((B, OUT), full),
        ],
        out_shape=[
            jax.ShapeDtypeStruct((B, H), jnp.float32),
            jax.ShapeDtypeStruct((B, H), jnp.float32),
            jax.ShapeDtypeStruct((B, OUT), jnp.float32),
        ],
        scratch_shapes=[
            pltpu.VMEM((B, H), jnp.float32),
            pltpu.VMEM((B, H), jnp.float32),
        ],
    )(x, wih_t, whh_t, bias, fcw_t, fcb, h0, c0)


def kernel(region_sequences, time_sequences, region_table, time_table,
           W_ih, W_hh, b_ih, b_hh, fc_W, fc_b):
    colscale = jnp.concatenate(
        [jnp.full((2 * H,), 0.5, jnp.float32),
         jnp.ones((H,), jnp.float32),
         jnp.full((H,), 0.5, jnp.float32)]
    )
    wih_t = (W_ih.T * colscale[None, :]).astype(jnp.bfloat16)
    whh_t = (W_hh.T * colscale[None, :]).astype(jnp.bfloat16)
    bias = ((b_ih + b_hh) * colscale).reshape(1, 4 * H)
    fcw_t = (fc_W.T * 0.5).astype(jnp.bfloat16)
    fcb = (fc_b * 0.5).reshape(1, OUT)

    seq_p = jnp.pad(region_sequences, ((0, 0), (0, 128 - L)))
    tsq_p = jnp.pad(time_sequences, ((0, 0), (0, 128 - L)))
    ctab = jnp.concatenate([region_table, time_table], axis=1)
    xs = [_sc_gather_chunk(c, seq_p, tsq_p, ctab).reshape(LC, B, D)
          for c in range(NCHUNK)]
    h = jnp.zeros((B, H), jnp.float32)
    c = jnp.zeros((B, H), jnp.float32)
    out = None
    for ci in range(NCHUNK):
        h, c, out = _lstm_tc(xs[ci], wih_t, whh_t, bias, fcw_t, fcb,
                             h, c, last=(ci == NCHUNK - 1))
    return out
